# Initial kernel scaffold; baseline (speedup 1.0000x reference)
import jax
import jax.numpy as jnp
from jax.experimental import pallas as pl


def kernel(x, emb_table, ln_w, ln_b):
    raise NotImplementedError("write your pallas kernel here")


# speedup vs baseline: 3.3428x; 3.3428x over previous
import functools

import jax
import jax.numpy as jnp
from jax import lax
from jax.experimental import pallas as pl
from jax.experimental.pallas import tpu as pltpu
from jax.experimental.pallas import tpu_sc as plsc

EMB_DIM = 100000
OUT_DIM = 64
BATCH = 4096
HIST = 200
LN_EPS = 1e-5

B_TOTAL = BATCH * HIST
NC, NS = 2, 16
NW = NC * NS
CH = 128
B_PER_W = B_TOTAL // NW
N_CH = B_PER_W // CH


def _idx_body(x_ref, o_ref):
    xb = x_ref[...]
    acc = xb[:, 0:8]
    for t in range(1, HIST // 8):
        acc = acc + xb[:, 8 * t:8 * t + 8]
    a = acc[:, 0:4] + acc[:, 4:8]
    b = a[:, 0:2] + a[:, 2:4]
    s = b[:, 0:1] + b[:, 1:2]
    o_ref[...] = ((xb / s) * float(EMB_DIM - 1)).astype(jnp.int32)


def _ln_table_body(t_ref, w_ref, b_ref, o_ref):
    t = t_ref[...]
    m = jnp.mean(t, axis=-1, keepdims=True)
    v = jnp.mean((t - m) ** 2, axis=-1, keepdims=True)
    o_ref[...] = ((t - m) / jnp.sqrt(v + LN_EPS)) * w_ref[...] + b_ref[...]


def _sc_gather(idx_hbm, table_hbm, out_hbm, idx_v, rows_v, gsem, osem):
    wid = lax.axis_index("s") * NC + lax.axis_index("c")
    base = wid * B_PER_W
    pltpu.sync_copy(idx_hbm.at[pl.ds(wid * N_CH, N_CH)], idx_v)

    def start_gather(j, buf):
        pltpu.async_copy(table_hbm.at[idx_v.at[j]], rows_v.at[buf], gsem)

    start_gather(0, 0)

    def body(j, _):
        buf = lax.rem(j, 2)

        @pl.when(j + 1 < N_CH)
        def _():
            start_gather(j + 1, 1 - buf)

        pltpu.make_async_copy(table_hbm.at[idx_v.at[j]], rows_v.at[buf],
                              gsem).wait()
        pltpu.async_copy(rows_v.at[buf],
                         out_hbm.at[pl.ds(base + j * CH, CH)], osem)
        pltpu.make_async_copy(rows_v.at[buf],
                              out_hbm.at[pl.ds(base + j * CH, CH)],
                              osem).wait()
        return 0

    lax.fori_loop(0, N_CH, body, 0)


def kernel(x, emb_table, ln_w, ln_b):
    idx = pl.pallas_call(
        _idx_body,
        out_shape=jax.ShapeDtypeStruct((BATCH, HIST), jnp.int32),
    )(x)

    nt = pl.pallas_call(
        _ln_table_body,
        grid=(10,),
        in_specs=[
            pl.BlockSpec((EMB_DIM // 10, OUT_DIM), lambda i: (i, 0)),
            pl.BlockSpec((1, OUT_DIM), lambda i: (0, 0)),
            pl.BlockSpec((1, OUT_DIM), lambda i: (0, 0)),
        ],
        out_specs=pl.BlockSpec((EMB_DIM // 10, OUT_DIM), lambda i: (i, 0)),
        out_shape=jax.ShapeDtypeStruct((EMB_DIM, OUT_DIM), jnp.float32),
    )(emb_table, ln_w.reshape(1, OUT_DIM), ln_b.reshape(1, OUT_DIM))

    mesh = plsc.VectorSubcoreMesh(core_axis_name="c", subcore_axis_name="s")
    gather = functools.partial(
        pl.kernel,
        mesh=mesh,
        compiler_params=pltpu.CompilerParams(use_tc_tiling_on_sc=False),
        out_type=jax.ShapeDtypeStruct((B_TOTAL, OUT_DIM), jnp.float32),
        scratch_types=[
            pltpu.VMEM((N_CH, CH), jnp.int32),
            pltpu.VMEM((2, CH, OUT_DIM), jnp.float32),
            pltpu.SemaphoreType.DMA,
            pltpu.SemaphoreType.DMA,
        ],
    )(_sc_gather)

    out_flat = gather(idx.reshape(B_TOTAL // CH, CH), nt)
    return out_flat.reshape(BATCH, HIST, OUT_DIM)


# speedup vs baseline: 3.3570x; 1.0042x over previous
import functools

import jax
import jax.numpy as jnp
from jax import lax
from jax.experimental import pallas as pl
from jax.experimental.pallas import tpu as pltpu
from jax.experimental.pallas import tpu_sc as plsc

EMB_DIM = 100000
OUT_DIM = 64
BATCH = 4096
HIST = 200
LN_EPS = 1e-5

B_TOTAL = BATCH * HIST
NC, NS = 2, 16
NW = NC * NS
CH = 128
B_PER_W = B_TOTAL // NW
N_CH = B_PER_W // CH


def _idx_body(x_ref, o_ref):
    xb = x_ref[...]
    acc = xb[:, 0:8]
    for t in range(1, HIST // 8):
        acc = acc + xb[:, 8 * t:8 * t + 8]
    a = acc[:, 0:4] + acc[:, 4:8]
    b = a[:, 0:2] + a[:, 2:4]
    s = b[:, 0:1] + b[:, 1:2]
    o_ref[...] = ((xb / s) * float(EMB_DIM - 1)).astype(jnp.int32)


def _ln_table_body(t_ref, w_ref, b_ref, o_ref):
    t = t_ref[...]
    m = jnp.mean(t, axis=-1, keepdims=True)
    v = jnp.mean((t - m) ** 2, axis=-1, keepdims=True)
    o_ref[...] = ((t - m) / jnp.sqrt(v + LN_EPS)) * w_ref[...] + b_ref[...]


K_CH = 4
G_ROWS = K_CH * CH
N_GRP = N_CH // K_CH


def _sc_gather(idx_hbm, table_hbm, out_hbm, idx_v, rows_v, gsem, osem):
    wid = lax.axis_index("s") * NC + lax.axis_index("c")
    base = wid * B_PER_W
    pltpu.sync_copy(idx_hbm.at[pl.ds(wid * N_CH, N_CH)], idx_v)

    def fire_group(g, b):
        for k in range(K_CH):
            pltpu.async_copy(table_hbm.at[idx_v.at[g * K_CH + k]],
                             rows_v.at[b, pl.ds(k * CH, CH)], gsem)

    def wait_group(g, b):
        for k in range(K_CH):
            pltpu.make_async_copy(table_hbm.at[idx_v.at[g * K_CH + k]],
                                  rows_v.at[b, pl.ds(k * CH, CH)],
                                  gsem).wait()

    def out_slice(g):
        return out_hbm.at[pl.ds(base + g * G_ROWS, G_ROWS)]

    fire_group(0, 0)

    def body(g, _):
        b = lax.rem(g, 2)

        @pl.when(g >= 1)
        def _():
            pltpu.make_async_copy(rows_v.at[1 - b], out_slice(g - 1),
                                  osem).wait()

        @pl.when(g + 1 < N_GRP)
        def _():
            fire_group(g + 1, 1 - b)

        wait_group(g, b)
        pltpu.async_copy(rows_v.at[b], out_slice(g), osem)
        return 0

    lax.fori_loop(0, N_GRP, body, 0)
    pltpu.make_async_copy(rows_v.at[(N_GRP - 1) % 2], out_slice(N_GRP - 1),
                          osem).wait()


def kernel(x, emb_table, ln_w, ln_b):
    idx = pl.pallas_call(
        _idx_body,
        out_shape=jax.ShapeDtypeStruct((BATCH, HIST), jnp.int32),
    )(x)

    nt = pl.pallas_call(
        _ln_table_body,
        grid=(10,),
        in_specs=[
            pl.BlockSpec((EMB_DIM // 10, OUT_DIM), lambda i: (i, 0)),
            pl.BlockSpec((1, OUT_DIM), lambda i: (0, 0)),
            pl.BlockSpec((1, OUT_DIM), lambda i: (0, 0)),
        ],
        out_specs=pl.BlockSpec((EMB_DIM // 10, OUT_DIM), lambda i: (i, 0)),
        out_shape=jax.ShapeDtypeStruct((EMB_DIM, OUT_DIM), jnp.float32),
    )(emb_table, ln_w.reshape(1, OUT_DIM), ln_b.reshape(1, OUT_DIM))

    mesh = plsc.VectorSubcoreMesh(core_axis_name="c", subcore_axis_name="s")
    gather = functools.partial(
        pl.kernel,
        mesh=mesh,
        compiler_params=pltpu.CompilerParams(use_tc_tiling_on_sc=False),
        out_type=jax.ShapeDtypeStruct((B_TOTAL, OUT_DIM), jnp.float32),
        scratch_types=[
            pltpu.VMEM((N_CH, CH), jnp.int32),
            pltpu.VMEM((2, G_ROWS, OUT_DIM), jnp.float32),
            pltpu.SemaphoreType.DMA,
            pltpu.SemaphoreType.DMA,
        ],
    )(_sc_gather)

    out_flat = gather(idx.reshape(B_TOTAL // CH, CH), nt)
    return out_flat.reshape(BATCH, HIST, OUT_DIM)
